# baseline (speedup 1.0000x reference)
import functools

import jax
import jax.numpy as jnp
from jax import lax
from jax.experimental import pallas as pl
from jax.experimental.pallas import tpu as pltpu
from jax.experimental.pallas import tpu_sc as plsc

N = 10000
NP = 10240
D = 128
E = 320000
CH = 128
NCHUNK = 2560
EPAD = NCHUNK * CH - E
NJUNK = NP - N
DEG_N = 10240

_f32 = jnp.float32
_MESH = plsc.VectorSubcoreMesh(core_axis_name="core", subcore_axis_name="subcore")


def _zero_rows(rows_v):
    def body(r, carry):
        for k in range(D // 16):
            rows_v[r, pl.ds(k * 16, 16)] = jnp.zeros((16,), _f32)
        return carry
    lax.fori_loop(0, CH, body, 0)


def _zero_acc(acc, rows_v, s):
    zb = pl.multiple_of(s * 640, 8)
    for i in range(5):
        pltpu.sync_copy(rows_v, acc.at[pl.ds(zb + i * 128, 128)])


def _writeback_acc(acc, rows_v, out_ref, s):
    @pl.when(s < 15)
    def _():
        zb = pl.multiple_of(s * 640, 8)
        for i in range(5):
            pltpu.sync_copy(acc.at[pl.ds(zb + i * 128, 128)], rows_v)
            pltpu.sync_copy(rows_v, out_ref.at[pl.ds(zb + i * 128, 128)])

    @pl.when(s == 15)
    def _():
        for i in range(3):
            pltpu.sync_copy(acc.at[pl.ds(9600 + i * 128, 128)], rows_v)
            pltpu.sync_copy(rows_v, out_ref.at[pl.ds(9600 + i * 128, 128)])
        pltpu.sync_copy(acc.at[pl.ds(9984, 16)], rows_v.at[pl.ds(0, 16)])
        pltpu.sync_copy(rows_v.at[pl.ds(0, 16)], out_ref.at[pl.ds(9984, 16)])


SEG = 40


def _edge_pass(tbl_hbm, src_hbm, dst_hbm, acc, src_v, dst_v, r0, r1,
               g0s, g1s, s0s, s1s, base, nseg):
    for seg in range(nseg):
        b = pl.multiple_of(base + seg * SEG, 8)
        pltpu.sync_copy(src_hbm.at[pl.ds(b, SEG)], src_v)
        pltpu.sync_copy(dst_hbm.at[pl.ds(b, SEG)], dst_v)
        pltpu.async_copy(tbl_hbm.at[src_v.at[0]], r0, g0s)
        pltpu.async_copy(tbl_hbm.at[src_v.at[1]], r1, g1s)

        def body(j, carry):
            i0 = 2 * j
            pltpu.make_async_copy(tbl_hbm.at[src_v.at[i0]], r0, g0s).wait()
            pltpu.async_copy(r0, acc.at[dst_v.at[i0]], s0s, add=True)
            pltpu.make_async_copy(tbl_hbm.at[src_v.at[i0 + 1]], r1, g1s).wait()
            pltpu.async_copy(r1, acc.at[dst_v.at[i0 + 1]], s1s, add=True)
            pltpu.make_async_copy(tbl_hbm.at[pl.ds(0, CH)], r0, s0s).wait()

            @pl.when(i0 + 2 < SEG)
            def _():
                pltpu.async_copy(tbl_hbm.at[src_v.at[i0 + 2]], r0, g0s)
            pltpu.make_async_copy(tbl_hbm.at[pl.ds(0, CH)], r1, s1s).wait()

            @pl.when(i0 + 3 < SEG)
            def _():
                pltpu.async_copy(tbl_hbm.at[src_v.at[i0 + 3]], r1, g1s)
            return carry
        lax.fori_loop(0, SEG // 2, body, 0)


@functools.partial(
    pl.kernel,
    out_type=[jax.ShapeDtypeStruct((2, 1, DEG_N), _f32) for _ in range(6)],
    mesh=_MESH,
    scratch_types=[
        pltpu.VMEM((80, CH), jnp.int32),
        pltpu.VMEM((CH,), _f32),
        pltpu.VMEM((640,), _f32),
        pltpu.VMEM((640,), _f32),
    ] + [pltpu.VMEM_SHARED((DEG_N,), _f32) for _ in range(6)],
)
def _deg_kernel(idx_hbm, o0, o1, o2, o3, o4, o5,
                slab, ones_v, zer_v, bnc_v, a0, a1, a2, a3, a4, a5):
    outs = (o0, o1, o2, o3, o4, o5)
    accs = (a0, a1, a2, a3, a4, a5)
    c = lax.axis_index("core")
    s = lax.axis_index("subcore")
    wid = c * 16 + s
    for k in range(CH // 16):
        ones_v[pl.ds(k * 16, 16)] = jnp.ones((16,), _f32)
    for k in range(640 // 16):
        zer_v[pl.ds(k * 16, 16)] = jnp.zeros((16,), _f32)
    zd = pl.multiple_of(s * 640, 8)
    for a in accs:
        pltpu.sync_copy(zer_v, a.at[pl.ds(zd, 640)])
    plsc.subcore_barrier()
    base = pl.multiple_of(wid * 80, 8)
    for jj, a in enumerate(accs):
        pltpu.sync_copy(idx_hbm.at[jj, pl.ds(base, 80)], slab)

        def body(g, carry, a=a):
            pltpu.sync_copy(ones_v, a.at[slab.at[g]], add=True)
            return carry
        lax.fori_loop(0, 80, body, 0)
    plsc.subcore_barrier()
    for jj, (a, o) in enumerate(zip(accs, outs)):
        pltpu.sync_copy(a.at[pl.ds(zd, 640)], bnc_v)
        pltpu.sync_copy(bnc_v, o.at[c, 0, pl.ds(zd, 640)])


@functools.partial(
    pl.kernel,
    out_type=[jax.ShapeDtypeStruct((N, D), _f32),
              jax.ShapeDtypeStruct((N, D), _f32)],
    mesh=_MESH,
    scratch_types=[
        pltpu.VMEM((SEG, CH), jnp.int32),
        pltpu.VMEM((SEG, CH), jnp.int32),
        pltpu.VMEM((CH, D), _f32),
        pltpu.VMEM((CH, D), _f32),
        pltpu.VMEM_SHARED((NP, D), _f32),
        pltpu.SemaphoreType.DMA,
        pltpu.SemaphoreType.DMA,
        pltpu.SemaphoreType.DMA,
        pltpu.SemaphoreType.DMA,
    ],
)
def _pass12_kernel(xs1_hbm, xs2_hbm, s1_hbm, d1_hbm, s2_hbm, d2_hbm,
                   p1_hbm, p2_hbm, src_v, dst_v, r0, r1, acc, g0s, g1s, s0s, s1s):
    c = lax.axis_index("core")
    s = lax.axis_index("subcore")
    _zero_rows(r0)
    _zero_acc(acc, r0, s)
    plsc.subcore_barrier()
    base = pl.multiple_of(s * 160, 8)

    @pl.when(c == 0)
    def _():
        _edge_pass(xs1_hbm, s1_hbm, d1_hbm, acc, src_v, dst_v, r0, r1,
                   g0s, g1s, s0s, s1s, base, 4)

    @pl.when(c == 1)
    def _():
        _edge_pass(xs2_hbm, s2_hbm, d2_hbm, acc, src_v, dst_v, r0, r1,
                   g0s, g1s, s0s, s1s, base, 4)

    plsc.subcore_barrier()

    @pl.when(c == 0)
    def _():
        _writeback_acc(acc, r0, p1_hbm, s)

    @pl.when(c == 1)
    def _():
        _writeback_acc(acc, r0, p2_hbm, s)


@functools.partial(
    pl.kernel,
    out_type=jax.ShapeDtypeStruct((2, N, D), _f32),
    mesh=_MESH,
    scratch_types=[
        pltpu.VMEM((SEG, CH), jnp.int32),
        pltpu.VMEM((SEG, CH), jnp.int32),
        pltpu.VMEM((CH, D), _f32),
        pltpu.VMEM((CH, D), _f32),
        pltpu.VMEM_SHARED((NP, D), _f32),
        pltpu.SemaphoreType.DMA,
        pltpu.SemaphoreType.DMA,
        pltpu.SemaphoreType.DMA,
        pltpu.SemaphoreType.DMA,
    ],
)
def _pass0_kernel(xs0a_hbm, xs0b_hbm, s0_hbm, d0_hbm, out_hbm, src_v, dst_v,
                  r0, r1, acc, g0s, g1s, s0s, s1s):
    c = lax.axis_index("core")
    s = lax.axis_index("subcore")
    _zero_rows(r0)
    _zero_acc(acc, r0, s)
    plsc.subcore_barrier()
    wid = c * 16 + s
    base = pl.multiple_of(wid * 80, 8)

    @pl.when(c == 0)
    def _():
        _edge_pass(xs0a_hbm, s0_hbm, d0_hbm, acc, src_v, dst_v, r0, r1,
                   g0s, g1s, s0s, s1s, base, 2)

    @pl.when(c == 1)
    def _():
        _edge_pass(xs0b_hbm, s0_hbm, d0_hbm, acc, src_v, dst_v, r0, r1,
                   g0s, g1s, s0s, s1s, base, 2)
    plsc.subcore_barrier()
    _writeback_acc(acc, r0, out_hbm.at[c], s)


def _scales_body(d0, d1, d2, d3, d4, d5, out_ref):
    for j, d in enumerate((d0, d1, d2, d3, d4, d5)):
        out_ref[j] = lax.rsqrt(jnp.maximum(d[0, 0] + d[1, 0], 1.0))


def _xs_body(x_ref, s1_ref, s2_ref, o1_ref, o2_ref):
    x = x_ref[...]
    zpad = jnp.zeros((NP - N, D), _f32)
    o1_ref[pl.ds(0, N), :] = x * s1_ref[...]
    o1_ref[pl.ds(N, NP - N), :] = zpad
    o2_ref[pl.ds(0, N), :] = x * s2_ref[...]
    o2_ref[pl.ds(N, NP - N), :] = zpad


def _mid_body(p1_ref, w11_ref, b11_ref, w21_ref, b21_ref, sd1_ref, ss0_ref,
              oc_ref, xs0a_ref, xs0b_ref):
    U = p1_ref[...]
    d1 = sd1_ref[...]
    h = jnp.maximum(jnp.dot(U, w11_ref[...], preferred_element_type=_f32) * d1
                    + b11_ref[...], 0.0)
    oc_ref[...] = (jnp.dot(U, w21_ref[...], preferred_element_type=_f32) * d1
                   + b21_ref[...])
    xs0 = h * ss0_ref[...]
    zp = jnp.zeros((NP - N, D), _f32)
    xs0a_ref[pl.ds(0, N), :] = xs0
    xs0a_ref[pl.ds(N, NP - N), :] = zp
    xs0b_ref[pl.ds(0, N), :] = xs0
    xs0b_ref[pl.ds(N, NP - N), :] = zp


def _out_body(p0_ref, p2_ref, w20_ref, b20_ref, w22_ref, b22_ref,
              sd0_ref, sd2_ref, o_ref):
    p0 = p0_ref[0] + p0_ref[1]
    o_ref[...] = (jnp.dot(p0, w20_ref[...], preferred_element_type=_f32)
                  * sd0_ref[...] + b20_ref[...]
                  + jnp.dot(p2_ref[...], w22_ref[...], preferred_element_type=_f32)
                  * sd2_ref[...] + b22_ref[...])


def kernel(x_loans, x_clients, edge_rel0, edge_rel1, edge_rel2,
           W1_0, b1_0, W1_1, b1_1, W1_2, b1_2,
           W2_0, b2_0, W2_1, b2_1, W2_2, b2_2):
    padv = N + (jnp.arange(EPAD, dtype=jnp.int32) % NJUNK)

    def chunks(v):
        return jnp.concatenate([v, padv]).reshape(NCHUNK, CH)

    s0, d0 = chunks(edge_rel0[0]), chunks(edge_rel0[1])
    s1, d1 = chunks(edge_rel1[0]), chunks(edge_rel1[1])
    s2, d2 = chunks(edge_rel2[0]), chunks(edge_rel2[1])
    idx6 = jnp.stack([s0, d0, s1, d1, s2, d2])

    degp = _deg_kernel(idx6)
    scal = pl.pallas_call(
        _scales_body,
        out_shape=jax.ShapeDtypeStruct((6, DEG_N), _f32))(*degp)
    ss0, sd0, ss1, sd1, ss2, sd2 = (scal[j, :N].reshape(N, 1) for j in range(6))

    xs1, xs2 = pl.pallas_call(
        _xs_body,
        out_shape=[jax.ShapeDtypeStruct((NP, D), _f32)] * 2)(x_loans, ss1, ss2)

    P1, P2 = _pass12_kernel(xs1, xs2, s1, d1, s2, d2)

    out_clients, xs0a, xs0b = pl.pallas_call(
        _mid_body,
        out_shape=[jax.ShapeDtypeStruct((N, D), _f32),
                   jax.ShapeDtypeStruct((NP, D), _f32),
                   jax.ShapeDtypeStruct((NP, D), _f32)])(
            P1, W1_1, b1_1.reshape(1, D), W2_1, b2_1.reshape(1, D), sd1, ss0)

    P0p = _pass0_kernel(xs0a, xs0b, s0, d0)

    out_loans = pl.pallas_call(
        _out_body,
        out_shape=jax.ShapeDtypeStruct((N, D), _f32))(
            P0p, P2, W2_0, b2_0.reshape(1, D), W2_2, b2_2.reshape(1, D), sd0, sd2)

    return (out_loans, out_clients)


# speedup vs baseline: 1.2457x; 1.2457x over previous
import functools

import jax
import jax.numpy as jnp
from jax import lax
from jax.experimental import pallas as pl
from jax.experimental.pallas import tpu as pltpu
from jax.experimental.pallas import tpu_sc as plsc

N = 10000
NP = 10240
D = 128
E = 320000
CH = 128
NCHUNK = 2560
EPAD = NCHUNK * CH - E
NJUNK = NP - N
DEG_N = 10240

_f32 = jnp.float32
_MESH = plsc.VectorSubcoreMesh(core_axis_name="core", subcore_axis_name="subcore")


def _zero_rows(rows_v):
    def body(r, carry):
        for k in range(D // 16):
            rows_v[r, pl.ds(k * 16, 16)] = jnp.zeros((16,), _f32)
        return carry
    lax.fori_loop(0, CH, body, 0)


def _zero_acc(acc, rows_v, s):
    zb = pl.multiple_of(s * 640, 8)
    for i in range(5):
        pltpu.sync_copy(rows_v, acc.at[pl.ds(zb + i * 128, 128)])


def _writeback_acc(acc, rows_v, out_ref, s):
    @pl.when(s < 15)
    def _():
        zb = pl.multiple_of(s * 640, 8)
        for i in range(5):
            pltpu.sync_copy(acc.at[pl.ds(zb + i * 128, 128)], rows_v)
            pltpu.sync_copy(rows_v, out_ref.at[pl.ds(zb + i * 128, 128)])

    @pl.when(s == 15)
    def _():
        for i in range(3):
            pltpu.sync_copy(acc.at[pl.ds(9600 + i * 128, 128)], rows_v)
            pltpu.sync_copy(rows_v, out_ref.at[pl.ds(9600 + i * 128, 128)])
        pltpu.sync_copy(acc.at[pl.ds(9984, 16)], rows_v.at[pl.ds(0, 16)])
        pltpu.sync_copy(rows_v.at[pl.ds(0, 16)], out_ref.at[pl.ds(9984, 16)])


SEG = 40


def _edge_pass(tbl_hbm, src_hbm, dst_hbm, acc, src_v, dst_v, r0, r1,
               g0s, g1s, base, segs):
    off = 0
    for L in segs:
        b = pl.multiple_of(base + off, 8)
        off += L
        pltpu.sync_copy(src_hbm.at[pl.ds(b, L)], src_v.at[pl.ds(0, L)])
        pltpu.sync_copy(dst_hbm.at[pl.ds(b, L)], dst_v.at[pl.ds(0, L)])
        pltpu.async_copy(tbl_hbm.at[src_v.at[0]], r0, g0s)

        def body(j, carry, L=L):
            i0 = 2 * j
            pltpu.async_copy(tbl_hbm.at[src_v.at[i0 + 1]], r1, g1s)
            pltpu.make_async_copy(tbl_hbm.at[src_v.at[i0]], r0, g0s).wait()
            pltpu.sync_copy(r0, acc.at[dst_v.at[i0]], add=True)

            @pl.when(i0 + 2 < L)
            def _():
                pltpu.async_copy(tbl_hbm.at[src_v.at[i0 + 2]], r0, g0s)
            pltpu.make_async_copy(tbl_hbm.at[src_v.at[i0 + 1]], r1, g1s).wait()
            pltpu.sync_copy(r1, acc.at[dst_v.at[i0 + 1]], add=True)
            return carry
        lax.fori_loop(0, L // 2, body, 0)


@functools.partial(
    pl.kernel,
    out_type=[jax.ShapeDtypeStruct((2, 1, DEG_N), _f32) for _ in range(6)],
    mesh=_MESH,
    scratch_types=[
        pltpu.VMEM((80, CH), jnp.int32),
        pltpu.VMEM((CH,), _f32),
        pltpu.VMEM((640,), _f32),
        pltpu.VMEM((640,), _f32),
    ] + [pltpu.VMEM_SHARED((DEG_N,), _f32) for _ in range(6)],
)
def _deg_kernel(i0_hbm, i1_hbm, i2_hbm, i3_hbm, i4_hbm, i5_hbm,
                o0, o1, o2, o3, o4, o5,
                slab, ones_v, zer_v, bnc_v, a0, a1, a2, a3, a4, a5):
    idxs = (i0_hbm, i1_hbm, i2_hbm, i3_hbm, i4_hbm, i5_hbm)
    outs = (o0, o1, o2, o3, o4, o5)
    accs = (a0, a1, a2, a3, a4, a5)
    c = lax.axis_index("core")
    s = lax.axis_index("subcore")
    wid = c * 16 + s
    for k in range(CH // 16):
        ones_v[pl.ds(k * 16, 16)] = jnp.ones((16,), _f32)
    for k in range(640 // 16):
        zer_v[pl.ds(k * 16, 16)] = jnp.zeros((16,), _f32)
    zd = pl.multiple_of(s * 640, 8)
    for a in accs:
        pltpu.sync_copy(zer_v, a.at[pl.ds(zd, 640)])
    plsc.subcore_barrier()

    base = pl.multiple_of(wid * 80, 8)
    for ih, a in zip(idxs, accs):
        pltpu.sync_copy(ih.at[pl.ds(base, 80)], slab)

        def body(g, carry, a=a):
            pltpu.sync_copy(ones_v, a.at[slab.at[g]], add=True)
            return carry
        lax.fori_loop(0, 80, body, 0)
    plsc.subcore_barrier()
    for a, o in zip(accs, outs):
        pltpu.sync_copy(a.at[pl.ds(zd, 640)], bnc_v)
        pltpu.sync_copy(bnc_v, o.at[c, 0, pl.ds(zd, 640)])


_SC_PASS_SCRATCH = [
    pltpu.VMEM((SEG, CH), jnp.int32),
    pltpu.VMEM((SEG, CH), jnp.int32),
    pltpu.VMEM((CH, D), _f32),
    pltpu.VMEM((CH, D), _f32),
    pltpu.VMEM_SHARED((NP, D), _f32),
    pltpu.SemaphoreType.DMA,
    pltpu.SemaphoreType.DMA,
]


@functools.partial(
    pl.kernel,
    out_type=[jax.ShapeDtypeStruct((N, D), _f32),
              jax.ShapeDtypeStruct((N, D), _f32)],
    mesh=_MESH,
    scratch_types=list(_SC_PASS_SCRATCH),
)
def _pass12_kernel(xs1_hbm, xs2_hbm, s1_hbm, d1_hbm, s2_hbm, d2_hbm,
                   p1_hbm, p2_hbm, src_v, dst_v, r0, r1, acc, g0s, g1s):
    c = lax.axis_index("core")
    s = lax.axis_index("subcore")
    _zero_rows(r0)
    _zero_acc(acc, r0, s)
    plsc.subcore_barrier()

    @pl.when(c == 0)
    def _():
        _edge_pass(xs1_hbm, s1_hbm, d1_hbm, acc, src_v, dst_v, r0, r1,
                   g0s, g1s, pl.multiple_of(s * 160, 8), (SEG, SEG, SEG, SEG))

    @pl.when(c == 1)
    def _():
        _edge_pass(xs2_hbm, s2_hbm, d2_hbm, acc, src_v, dst_v, r0, r1,
                   g0s, g1s, pl.multiple_of(s * 160, 8), (SEG, SEG, SEG, SEG))

    plsc.subcore_barrier()

    @pl.when(c == 0)
    def _():
        _writeback_acc(acc, r0, p1_hbm, s)

    @pl.when(c == 1)
    def _():
        _writeback_acc(acc, r0, p2_hbm, s)


@functools.partial(
    pl.kernel,
    out_type=jax.ShapeDtypeStruct((2, N, D), _f32),
    mesh=_MESH,
    scratch_types=list(_SC_PASS_SCRATCH),
)
def _pass0_kernel(xs0a_hbm, xs0b_hbm, s0_hbm, d0_hbm, out_hbm, src_v, dst_v,
                  r0, r1, acc, g0s, g1s):
    c = lax.axis_index("core")
    s = lax.axis_index("subcore")
    _zero_rows(r0)
    _zero_acc(acc, r0, s)
    plsc.subcore_barrier()
    wid = c * 16 + s

    @pl.when(c == 0)
    def _():
        _edge_pass(xs0a_hbm, s0_hbm, d0_hbm, acc, src_v, dst_v, r0, r1,
                   g0s, g1s, pl.multiple_of(wid * 80, 8), (SEG, SEG))

    @pl.when(c == 1)
    def _():
        _edge_pass(xs0b_hbm, s0_hbm, d0_hbm, acc, src_v, dst_v, r0, r1,
                   g0s, g1s, pl.multiple_of(wid * 80, 8), (SEG, SEG))

    plsc.subcore_barrier()
    _writeback_acc(acc, r0, out_hbm.at[c], s)


def _scales_body(d0, d1, d2, d3, d4, d5, out_ref):
    for j, d in enumerate((d0, d1, d2, d3, d4, d5)):
        out_ref[j] = lax.rsqrt(jnp.maximum(d[0, 0] + d[1, 0], 1.0))


def _xs_body(x_ref, s1_ref, s2_ref, o1_ref, o2_ref):
    x = x_ref[...]
    zp = jnp.zeros((NP - N, D), _f32)
    o1_ref[pl.ds(0, N), :] = x * s1_ref[...]
    o1_ref[pl.ds(N, NP - N), :] = zp
    o2_ref[pl.ds(0, N), :] = x * s2_ref[...]
    o2_ref[pl.ds(N, NP - N), :] = zp


def _mid_body(p1_ref, w11_ref, b11_ref, w21_ref, b21_ref, sd1_ref, ss0_ref,
              oc_ref, xs0a_ref, xs0b_ref):
    U = p1_ref[...]
    d1 = sd1_ref[...]
    h = jnp.maximum(jnp.dot(U, w11_ref[...], preferred_element_type=_f32) * d1
                    + b11_ref[...], 0.0)
    oc_ref[...] = (jnp.dot(U, w21_ref[...], preferred_element_type=_f32) * d1
                   + b21_ref[...])
    xs0 = h * ss0_ref[...]
    zp = jnp.zeros((NP - N, D), _f32)
    xs0a_ref[pl.ds(0, N), :] = xs0
    xs0a_ref[pl.ds(N, NP - N), :] = zp
    xs0b_ref[pl.ds(0, N), :] = xs0
    xs0b_ref[pl.ds(N, NP - N), :] = zp


def _out_body(p0_ref, p2_ref, w20_ref, b20_ref, w22_ref, b22_ref,
              sd0_ref, sd2_ref, o_ref):
    p0 = p0_ref[0] + p0_ref[1]
    o_ref[...] = (jnp.dot(p0, w20_ref[...], preferred_element_type=_f32)
                  * sd0_ref[...] + b20_ref[...]
                  + jnp.dot(p2_ref[...], w22_ref[...], preferred_element_type=_f32)
                  * sd2_ref[...] + b22_ref[...])


def kernel(x_loans, x_clients, edge_rel0, edge_rel1, edge_rel2,
           W1_0, b1_0, W1_1, b1_1, W1_2, b1_2,
           W2_0, b2_0, W2_1, b2_1, W2_2, b2_2):
    padv = N + (jnp.arange(EPAD, dtype=jnp.int32) % NJUNK)

    def chunks(v):
        return jnp.concatenate([v, padv]).reshape(NCHUNK, CH)

    s0, d0 = chunks(edge_rel0[0]), chunks(edge_rel0[1])
    s1, d1 = chunks(edge_rel1[0]), chunks(edge_rel1[1])
    s2, d2 = chunks(edge_rel2[0]), chunks(edge_rel2[1])

    degp = _deg_kernel(s0, d0, s1, d1, s2, d2)
    scal = pl.pallas_call(
        _scales_body,
        out_shape=jax.ShapeDtypeStruct((6, DEG_N), _f32))(*degp)
    ss0, sd0, ss1, sd1, ss2, sd2 = (scal[j, :N].reshape(N, 1) for j in range(6))

    xs1, xs2 = pl.pallas_call(
        _xs_body,
        out_shape=[jax.ShapeDtypeStruct((NP, D), _f32)] * 2)(x_loans, ss1, ss2)

    P1, P2 = _pass12_kernel(xs1, xs2, s1, d1, s2, d2)

    out_clients, xs0a, xs0b = pl.pallas_call(
        _mid_body,
        out_shape=[jax.ShapeDtypeStruct((N, D), _f32),
                   jax.ShapeDtypeStruct((NP, D), _f32),
                   jax.ShapeDtypeStruct((NP, D), _f32)])(
            P1, W1_1, b1_1.reshape(1, D), W2_1, b2_1.reshape(1, D), sd1, ss0)

    P0p = _pass0_kernel(xs0a, xs0b, s0, d0)

    out_loans = pl.pallas_call(
        _out_body,
        out_shape=jax.ShapeDtypeStruct((N, D), _f32))(
            P0p, P2, W2_0, b2_0.reshape(1, D), W2_2, b2_2.reshape(1, D), sd0, sd2)

    return (out_loans, out_clients)


# speedup vs baseline: 1.3251x; 1.0637x over previous
import functools

import jax
import jax.numpy as jnp
from jax import lax
from jax.experimental import pallas as pl
from jax.experimental.pallas import tpu as pltpu
from jax.experimental.pallas import tpu_sc as plsc

N = 10000
NP = 10240
D = 128
E = 320000
CH = 128
NCHUNK = 2560
EPAD = NCHUNK * CH - E
NJUNK = NP - N
DEG_N = 10240

_f32 = jnp.float32
_MESH = plsc.VectorSubcoreMesh(core_axis_name="core", subcore_axis_name="subcore")


def _zero_rows(rows_v):
    def body(r, carry):
        for k in range(D // 16):
            rows_v[r, pl.ds(k * 16, 16)] = jnp.zeros((16,), _f32)
        return carry
    lax.fori_loop(0, CH, body, 0)


def _zero_acc(acc, rows_v, s):
    zb = pl.multiple_of(s * 640, 8)
    for i in range(5):
        pltpu.sync_copy(rows_v, acc.at[pl.ds(zb + i * 128, 128)])


def _writeback_acc(acc, rows_v, out_ref, s):
    @pl.when(s < 15)
    def _():
        zb = pl.multiple_of(s * 640, 8)
        for i in range(5):
            pltpu.sync_copy(acc.at[pl.ds(zb + i * 128, 128)], rows_v)
            pltpu.sync_copy(rows_v, out_ref.at[pl.ds(zb + i * 128, 128)])

    @pl.when(s == 15)
    def _():
        for i in range(3):
            pltpu.sync_copy(acc.at[pl.ds(9600 + i * 128, 128)], rows_v)
            pltpu.sync_copy(rows_v, out_ref.at[pl.ds(9600 + i * 128, 128)])
        pltpu.sync_copy(acc.at[pl.ds(9984, 16)], rows_v.at[pl.ds(0, 16)])
        pltpu.sync_copy(rows_v.at[pl.ds(0, 16)], out_ref.at[pl.ds(9984, 16)])


SEG = 40


def _edge_pass(tbl_hbm, src_hbm, dst_hbm, acc, src_v, dst_v, r0, r1,
               g0s, g1s, base, segs):
    off = 0
    for L in segs:
        b = pl.multiple_of(base + off, 8)
        off += L
        pltpu.sync_copy(src_hbm.at[pl.ds(b, L)], src_v.at[pl.ds(0, L)])
        pltpu.sync_copy(dst_hbm.at[pl.ds(b, L)], dst_v.at[pl.ds(0, L)])
        pltpu.async_copy(tbl_hbm.at[src_v.at[0]], r0, g0s)

        def body(j, carry, L=L):
            i0 = 2 * j
            pltpu.async_copy(tbl_hbm.at[src_v.at[i0 + 1]], r1, g1s)
            pltpu.make_async_copy(tbl_hbm.at[src_v.at[i0]], r0, g0s).wait()
            pltpu.sync_copy(r0, acc.at[dst_v.at[i0]], add=True)

            @pl.when(i0 + 2 < L)
            def _():
                pltpu.async_copy(tbl_hbm.at[src_v.at[i0 + 2]], r0, g0s)
            pltpu.make_async_copy(tbl_hbm.at[src_v.at[i0 + 1]], r1, g1s).wait()
            pltpu.sync_copy(r1, acc.at[dst_v.at[i0 + 1]], add=True)
            return carry
        lax.fori_loop(0, L // 2, body, 0)


@functools.partial(
    pl.kernel,
    out_type=[jax.ShapeDtypeStruct((2, 1, DEG_N), _f32) for _ in range(6)],
    mesh=_MESH,
    scratch_types=[
        pltpu.VMEM((80, CH), jnp.int32),
        pltpu.VMEM((CH,), _f32),
        pltpu.VMEM((640,), _f32),
        pltpu.VMEM((640,), _f32),
        pltpu.SemaphoreType.DMA,
    ] + [pltpu.VMEM_SHARED((DEG_N,), _f32) for _ in range(6)],
)
def _deg_kernel(i0_hbm, i1_hbm, i2_hbm, i3_hbm, i4_hbm, i5_hbm,
                o0, o1, o2, o3, o4, o5,
                slab, ones_v, zer_v, bnc_v, dsem, a0, a1, a2, a3, a4, a5):
    idxs = (i0_hbm, i1_hbm, i2_hbm, i3_hbm, i4_hbm, i5_hbm)
    outs = (o0, o1, o2, o3, o4, o5)
    accs = (a0, a1, a2, a3, a4, a5)
    c = lax.axis_index("core")
    s = lax.axis_index("subcore")
    wid = c * 16 + s
    for k in range(CH // 16):
        ones_v[pl.ds(k * 16, 16)] = jnp.ones((16,), _f32)
    for k in range(640 // 16):
        zer_v[pl.ds(k * 16, 16)] = jnp.zeros((16,), _f32)
    zd = pl.multiple_of(s * 640, 8)
    for a in accs:
        pltpu.sync_copy(zer_v, a.at[pl.ds(zd, 640)])
    plsc.subcore_barrier()

    base = pl.multiple_of(wid * 80, 8)
    for ih, a in zip(idxs, accs):
        pltpu.sync_copy(ih.at[pl.ds(base, 80)], slab)

        def body(grp, carry, a=a):
            g0 = 8 * grp
            for k in range(8):
                pltpu.async_copy(ones_v, a.at[slab.at[g0 + k]], dsem, add=True)
            for k in range(8):
                pltpu.make_async_copy(ones_v, a.at[slab.at[g0 + k]], dsem).wait()
            return carry
        lax.fori_loop(0, 10, body, 0)
    plsc.subcore_barrier()
    for a, o in zip(accs, outs):
        pltpu.sync_copy(a.at[pl.ds(zd, 640)], bnc_v)
        pltpu.sync_copy(bnc_v, o.at[c, 0, pl.ds(zd, 640)])


_SC_PASS_SCRATCH = [
    pltpu.VMEM((SEG, CH), jnp.int32),
    pltpu.VMEM((SEG, CH), jnp.int32),
    pltpu.VMEM((CH, D), _f32),
    pltpu.VMEM((CH, D), _f32),
    pltpu.VMEM_SHARED((NP, D), _f32),
    pltpu.SemaphoreType.DMA,
    pltpu.SemaphoreType.DMA,
]


@functools.partial(
    pl.kernel,
    out_type=[jax.ShapeDtypeStruct((N, D), _f32),
              jax.ShapeDtypeStruct((N, D), _f32)],
    mesh=_MESH,
    scratch_types=list(_SC_PASS_SCRATCH),
)
def _pass12_kernel(xs1_hbm, xs2_hbm, s1_hbm, d1_hbm, s2_hbm, d2_hbm,
                   p1_hbm, p2_hbm, src_v, dst_v, r0, r1, acc, g0s, g1s):
    c = lax.axis_index("core")
    s = lax.axis_index("subcore")
    _zero_rows(r0)
    _zero_acc(acc, r0, s)
    plsc.subcore_barrier()

    @pl.when(c == 0)
    def _():
        _edge_pass(xs1_hbm, s1_hbm, d1_hbm, acc, src_v, dst_v, r0, r1,
                   g0s, g1s, pl.multiple_of(s * 160, 8), (SEG, SEG, SEG, SEG))

    @pl.when(c == 1)
    def _():
        _edge_pass(xs2_hbm, s2_hbm, d2_hbm, acc, src_v, dst_v, r0, r1,
                   g0s, g1s, pl.multiple_of(s * 160, 8), (SEG, SEG, SEG, SEG))

    plsc.subcore_barrier()

    @pl.when(c == 0)
    def _():
        _writeback_acc(acc, r0, p1_hbm, s)

    @pl.when(c == 1)
    def _():
        _writeback_acc(acc, r0, p2_hbm, s)


@functools.partial(
    pl.kernel,
    out_type=jax.ShapeDtypeStruct((2, N, D), _f32),
    mesh=_MESH,
    scratch_types=list(_SC_PASS_SCRATCH),
)
def _pass0_kernel(xs0a_hbm, xs0b_hbm, s0_hbm, d0_hbm, out_hbm, src_v, dst_v,
                  r0, r1, acc, g0s, g1s):
    c = lax.axis_index("core")
    s = lax.axis_index("subcore")
    _zero_rows(r0)
    _zero_acc(acc, r0, s)
    plsc.subcore_barrier()
    wid = c * 16 + s

    @pl.when(c == 0)
    def _():
        _edge_pass(xs0a_hbm, s0_hbm, d0_hbm, acc, src_v, dst_v, r0, r1,
                   g0s, g1s, pl.multiple_of(wid * 80, 8), (SEG, SEG))

    @pl.when(c == 1)
    def _():
        _edge_pass(xs0b_hbm, s0_hbm, d0_hbm, acc, src_v, dst_v, r0, r1,
                   g0s, g1s, pl.multiple_of(wid * 80, 8), (SEG, SEG))

    plsc.subcore_barrier()
    _writeback_acc(acc, r0, out_hbm.at[c], s)


def _scales_body(d0, d1, d2, d3, d4, d5, out_ref):
    for j, d in enumerate((d0, d1, d2, d3, d4, d5)):
        out_ref[j] = lax.rsqrt(jnp.maximum(d[0, 0] + d[1, 0], 1.0))


def _xs_body(x_ref, s1_ref, s2_ref, o1_ref, o2_ref):
    x = x_ref[...]
    zp = jnp.zeros((NP - N, D), _f32)
    o1_ref[pl.ds(0, N), :] = x * s1_ref[...]
    o1_ref[pl.ds(N, NP - N), :] = zp
    o2_ref[pl.ds(0, N), :] = x * s2_ref[...]
    o2_ref[pl.ds(N, NP - N), :] = zp


def _mid_body(p1_ref, w11_ref, b11_ref, w21_ref, b21_ref, sd1_ref, ss0_ref,
              oc_ref, xs0a_ref, xs0b_ref):
    U = p1_ref[...]
    d1 = sd1_ref[...]
    h = jnp.maximum(jnp.dot(U, w11_ref[...], preferred_element_type=_f32) * d1
                    + b11_ref[...], 0.0)
    oc_ref[...] = (jnp.dot(U, w21_ref[...], preferred_element_type=_f32) * d1
                   + b21_ref[...])
    xs0 = h * ss0_ref[...]
    zp = jnp.zeros((NP - N, D), _f32)
    xs0a_ref[pl.ds(0, N), :] = xs0
    xs0a_ref[pl.ds(N, NP - N), :] = zp
    xs0b_ref[pl.ds(0, N), :] = xs0
    xs0b_ref[pl.ds(N, NP - N), :] = zp


def _out_body(p0_ref, p2_ref, w20_ref, b20_ref, w22_ref, b22_ref,
              sd0_ref, sd2_ref, o_ref):
    p0 = p0_ref[0] + p0_ref[1]
    o_ref[...] = (jnp.dot(p0, w20_ref[...], preferred_element_type=_f32)
                  * sd0_ref[...] + b20_ref[...]
                  + jnp.dot(p2_ref[...], w22_ref[...], preferred_element_type=_f32)
                  * sd2_ref[...] + b22_ref[...])


def kernel(x_loans, x_clients, edge_rel0, edge_rel1, edge_rel2,
           W1_0, b1_0, W1_1, b1_1, W1_2, b1_2,
           W2_0, b2_0, W2_1, b2_1, W2_2, b2_2):
    padv = N + (jnp.arange(EPAD, dtype=jnp.int32) % NJUNK)

    def chunks(v):
        return jnp.concatenate([v, padv]).reshape(NCHUNK, CH)

    s0, d0 = chunks(edge_rel0[0]), chunks(edge_rel0[1])
    s1, d1 = chunks(edge_rel1[0]), chunks(edge_rel1[1])
    s2, d2 = chunks(edge_rel2[0]), chunks(edge_rel2[1])

    degp = _deg_kernel(s0, d0, s1, d1, s2, d2)
    scal = pl.pallas_call(
        _scales_body,
        out_shape=jax.ShapeDtypeStruct((6, DEG_N), _f32))(*degp)
    ss0, sd0, ss1, sd1, ss2, sd2 = (scal[j, :N].reshape(N, 1) for j in range(6))

    xs1, xs2 = pl.pallas_call(
        _xs_body,
        out_shape=[jax.ShapeDtypeStruct((NP, D), _f32)] * 2)(x_loans, ss1, ss2)

    P1, P2 = _pass12_kernel(xs1, xs2, s1, d1, s2, d2)

    out_clients, xs0a, xs0b = pl.pallas_call(
        _mid_body,
        out_shape=[jax.ShapeDtypeStruct((N, D), _f32),
                   jax.ShapeDtypeStruct((NP, D), _f32),
                   jax.ShapeDtypeStruct((NP, D), _f32)])(
            P1, W1_1, b1_1.reshape(1, D), W2_1, b2_1.reshape(1, D), sd1, ss0)

    P0p = _pass0_kernel(xs0a, xs0b, s0, d0)

    out_loans = pl.pallas_call(
        _out_body,
        out_shape=jax.ShapeDtypeStruct((N, D), _f32))(
            P0p, P2, W2_0, b2_0.reshape(1, D), W2_2, b2_2.reshape(1, D), sd0, sd2)

    return (out_loans, out_clients)


# speedup vs baseline: 1.3989x; 1.0556x over previous
import functools

import jax
import jax.numpy as jnp
from jax import lax
from jax.experimental import pallas as pl
from jax.experimental.pallas import tpu as pltpu
from jax.experimental.pallas import tpu_sc as plsc

N = 10000
NP = 10240
D = 128
E = 320000
CH = 128
DEG_N = 10240
SEG = 40

_f32 = jnp.float32
_MESH = plsc.VectorSubcoreMesh(core_axis_name="core", subcore_axis_name="subcore")


def _zero_rows(rows_v):
    def body(r, carry):
        for k in range(D // 16):
            rows_v[r, pl.ds(k * 16, 16)] = jnp.zeros((16,), _f32)
        return carry
    lax.fori_loop(0, CH, body, 0)


def _zero_acc(acc, rows_v, s):
    zb = pl.multiple_of(s * 640, 8)
    for i in range(5):
        pltpu.sync_copy(rows_v, acc.at[pl.ds(zb + i * 128, 128)])


def _writeback_acc(acc, rows_v, out_ref, s):
    @pl.when(s < 15)
    def _():
        zb = pl.multiple_of(s * 640, 8)
        for i in range(5):
            pltpu.sync_copy(acc.at[pl.ds(zb + i * 128, 128)], rows_v)
            pltpu.sync_copy(rows_v, out_ref.at[pl.ds(zb + i * 128, 128)])

    @pl.when(s == 15)
    def _():
        for i in range(3):
            pltpu.sync_copy(acc.at[pl.ds(9600 + i * 128, 128)], rows_v)
            pltpu.sync_copy(rows_v, out_ref.at[pl.ds(9600 + i * 128, 128)])
        pltpu.sync_copy(acc.at[pl.ds(9984, 16)], rows_v.at[pl.ds(0, 16)])
        pltpu.sync_copy(rows_v.at[pl.ds(0, 16)], out_ref.at[pl.ds(9984, 16)])


def _idx(slab, g):
    return slab.at[pl.ds(pl.multiple_of(g * CH, CH), CH)]


def _edge_pass(tbl_hbm, edge_hbm, acc, src_v, dst_v, r0, r1,
               g0s, g1s, base, segs):
    off = 0
    for L in segs:
        ne = L * CH
        b = pl.multiple_of(base + off, CH)
        off += ne
        pltpu.sync_copy(edge_hbm.at[0, pl.ds(b, ne)], src_v.at[pl.ds(0, ne)])
        pltpu.sync_copy(edge_hbm.at[1, pl.ds(b, ne)], dst_v.at[pl.ds(0, ne)])
        pltpu.async_copy(tbl_hbm.at[_idx(src_v, 0)], r0, g0s)

        def body(j, carry, L=L):
            i0 = 2 * j
            pltpu.async_copy(tbl_hbm.at[_idx(src_v, i0 + 1)], r1, g1s)
            pltpu.make_async_copy(tbl_hbm.at[_idx(src_v, i0)], r0, g0s).wait()
            pltpu.sync_copy(r0, acc.at[_idx(dst_v, i0)], add=True)

            @pl.when(i0 + 2 < L)
            def _():
                pltpu.async_copy(tbl_hbm.at[_idx(src_v, i0 + 2)], r0, g0s)
            pltpu.make_async_copy(tbl_hbm.at[_idx(src_v, i0 + 1)], r1, g1s).wait()
            pltpu.sync_copy(r1, acc.at[_idx(dst_v, i0 + 1)], add=True)
            return carry
        lax.fori_loop(0, L // 2, body, 0)


@functools.partial(
    pl.kernel,
    out_type=[jax.ShapeDtypeStruct((2, 1, DEG_N), _f32) for _ in range(6)],
    mesh=_MESH,
    scratch_types=[
        pltpu.VMEM((80 * CH,), jnp.int32),
        pltpu.VMEM((CH,), _f32),
        pltpu.VMEM((640,), _f32),
        pltpu.VMEM((640,), _f32),
        pltpu.SemaphoreType.DMA,
    ] + [pltpu.VMEM_SHARED((DEG_N,), _f32) for _ in range(6)],
)
def _deg_kernel(e0_hbm, e1_hbm, e2_hbm, o0, o1, o2, o3, o4, o5,
                slab, ones_v, zer_v, bnc_v, dsem, a0, a1, a2, a3, a4, a5):
    jobs = ((e0_hbm, 0, a0, o0), (e0_hbm, 1, a1, o1),
            (e1_hbm, 0, a2, o2), (e1_hbm, 1, a3, o3),
            (e2_hbm, 0, a4, o4), (e2_hbm, 1, a5, o5))
    c = lax.axis_index("core")
    s = lax.axis_index("subcore")
    wid = c * 16 + s
    for k in range(CH // 16):
        ones_v[pl.ds(k * 16, 16)] = jnp.ones((16,), _f32)
    for k in range(640 // 16):
        zer_v[pl.ds(k * 16, 16)] = jnp.zeros((16,), _f32)
    zd = pl.multiple_of(s * 640, 8)
    for _eh, _row, a, _o in jobs:
        pltpu.sync_copy(zer_v, a.at[pl.ds(zd, 640)])
    plsc.subcore_barrier()

    base = pl.multiple_of(wid * (80 * CH), CH)

    def scat(a, nch):
        def body(grp, carry):
            g0 = 4 * grp
            for k in range(4):
                pltpu.async_copy(ones_v, a.at[_idx(slab, g0 + k)], dsem,
                                 add=True)
            for k in range(4):
                pltpu.make_async_copy(ones_v, a.at[_idx(slab, g0 + k)],
                                      dsem).wait()
            return carry
        lax.fori_loop(0, nch // 4, body, 0)

    @pl.when(wid < 31)
    def _():
        for eh, row, a, _o in jobs:
            pltpu.sync_copy(eh.at[row, pl.ds(base, 80 * CH)], slab)
            scat(a, 80)

    @pl.when(wid == 31)
    def _():
        for eh, row, a, _o in jobs:
            pltpu.sync_copy(eh.at[row, pl.ds(317440, 20 * CH)],
                            slab.at[pl.ds(0, 20 * CH)])
            scat(a, 20)

    plsc.subcore_barrier()
    for _eh, _row, a, o in jobs:
        pltpu.sync_copy(a.at[pl.ds(zd, 640)], bnc_v)
        pltpu.sync_copy(bnc_v, o.at[c, 0, pl.ds(zd, 640)])


_SC_PASS_SCRATCH = [
    pltpu.VMEM((SEG * CH,), jnp.int32),
    pltpu.VMEM((SEG * CH,), jnp.int32),
    pltpu.VMEM((CH, D), _f32),
    pltpu.VMEM((CH, D), _f32),
    pltpu.VMEM_SHARED((NP, D), _f32),
    pltpu.SemaphoreType.DMA,
    pltpu.SemaphoreType.DMA,
]


@functools.partial(
    pl.kernel,
    out_type=[jax.ShapeDtypeStruct((N, D), _f32),
              jax.ShapeDtypeStruct((N, D), _f32)],
    mesh=_MESH,
    scratch_types=list(_SC_PASS_SCRATCH),
)
def _pass12_kernel(xs1_hbm, xs2_hbm, e1_hbm, e2_hbm,
                   p1_hbm, p2_hbm, src_v, dst_v, r0, r1, acc, g0s, g1s):
    c = lax.axis_index("core")
    s = lax.axis_index("subcore")
    _zero_rows(r0)
    _zero_acc(acc, r0, s)
    plsc.subcore_barrier()

    def run(tbl, eh):
        @pl.when(s < 15)
        def _():
            _edge_pass(tbl, eh, acc, src_v, dst_v, r0, r1, g0s, g1s,
                       pl.multiple_of(s * (160 * CH), CH),
                       (SEG, SEG, SEG, SEG))

        @pl.when(s == 15)
        def _():
            _edge_pass(tbl, eh, acc, src_v, dst_v, r0, r1, g0s, g1s,
                       2400 * CH, (SEG, SEG, 20))

    @pl.when(c == 0)
    def _():
        run(xs1_hbm, e1_hbm)

    @pl.when(c == 1)
    def _():
        run(xs2_hbm, e2_hbm)

    plsc.subcore_barrier()

    @pl.when(c == 0)
    def _():
        _writeback_acc(acc, r0, p1_hbm, s)

    @pl.when(c == 1)
    def _():
        _writeback_acc(acc, r0, p2_hbm, s)


@functools.partial(
    pl.kernel,
    out_type=jax.ShapeDtypeStruct((2, N, D), _f32),
    mesh=_MESH,
    scratch_types=list(_SC_PASS_SCRATCH),
)
def _pass0_kernel(xs0a_hbm, xs0b_hbm, e0_hbm, out_hbm, src_v, dst_v,
                  r0, r1, acc, g0s, g1s):
    c = lax.axis_index("core")
    s = lax.axis_index("subcore")
    _zero_rows(r0)
    _zero_acc(acc, r0, s)
    plsc.subcore_barrier()
    wid = c * 16 + s

    def run(tbl):
        @pl.when(wid < 31)
        def _():
            _edge_pass(tbl, e0_hbm, acc, src_v, dst_v, r0, r1, g0s, g1s,
                       pl.multiple_of(wid * (80 * CH), CH), (SEG, SEG))

        @pl.when(wid == 31)
        def _():
            _edge_pass(tbl, e0_hbm, acc, src_v, dst_v, r0, r1, g0s, g1s,
                       317440, (20,))

    @pl.when(c == 0)
    def _():
        run(xs0a_hbm)

    @pl.when(c == 1)
    def _():
        run(xs0b_hbm)

    plsc.subcore_barrier()
    _writeback_acc(acc, r0, out_hbm.at[c], s)


def _scales_body(d0, d1, d2, d3, d4, d5, out_ref):
    for j, d in enumerate((d0, d1, d2, d3, d4, d5)):
        out_ref[j] = lax.rsqrt(jnp.maximum(d[0, 0] + d[1, 0], 1.0))


def _xs_body(x_ref, s1_ref, s2_ref, o1_ref, o2_ref):
    x = x_ref[...]
    o1_ref[...] = x * s1_ref[...]
    o2_ref[...] = x * s2_ref[...]


def _mid_body(p1_ref, w11_ref, b11_ref, w21_ref, b21_ref, sd1_ref, ss0_ref,
              oc_ref, xs0a_ref, xs0b_ref):
    U = p1_ref[...]
    d1 = sd1_ref[...]
    h = jnp.maximum(jnp.dot(U, w11_ref[...], preferred_element_type=_f32) * d1
                    + b11_ref[...], 0.0)
    oc_ref[...] = (jnp.dot(U, w21_ref[...], preferred_element_type=_f32) * d1
                   + b21_ref[...])
    xs0 = h * ss0_ref[...]
    xs0a_ref[...] = xs0
    xs0b_ref[...] = xs0


def _out_body(p0_ref, p2_ref, w20_ref, b20_ref, w22_ref, b22_ref,
              sd0_ref, sd2_ref, o_ref):
    p0 = p0_ref[0] + p0_ref[1]
    o_ref[...] = (jnp.dot(p0, w20_ref[...], preferred_element_type=_f32)
                  * sd0_ref[...] + b20_ref[...]
                  + jnp.dot(p2_ref[...], w22_ref[...], preferred_element_type=_f32)
                  * sd2_ref[...] + b22_ref[...])


def kernel(x_loans, x_clients, edge_rel0, edge_rel1, edge_rel2,
           W1_0, b1_0, W1_1, b1_1, W1_2, b1_2,
           W2_0, b2_0, W2_1, b2_1, W2_2, b2_2):
    degp = _deg_kernel(edge_rel0, edge_rel1, edge_rel2)
    scal = pl.pallas_call(
        _scales_body,
        out_shape=jax.ShapeDtypeStruct((6, DEG_N), _f32))(*degp)
    ss0, sd0, ss1, sd1, ss2, sd2 = (scal[j, :N].reshape(N, 1) for j in range(6))

    xs1, xs2 = pl.pallas_call(
        _xs_body,
        out_shape=[jax.ShapeDtypeStruct((N, D), _f32)] * 2)(x_loans, ss1, ss2)

    P1, P2 = _pass12_kernel(xs1, xs2, edge_rel1, edge_rel2)

    out_clients, xs0a, xs0b = pl.pallas_call(
        _mid_body,
        out_shape=[jax.ShapeDtypeStruct((N, D), _f32)] * 3)(
            P1, W1_1, b1_1.reshape(1, D), W2_1, b2_1.reshape(1, D), sd1, ss0)

    P0p = _pass0_kernel(xs0a, xs0b, edge_rel0)

    out_loans = pl.pallas_call(
        _out_body,
        out_shape=jax.ShapeDtypeStruct((N, D), _f32))(
            P0p, P2, W2_0, b2_0.reshape(1, D), W2_2, b2_2.reshape(1, D), sd0, sd2)

    return (out_loans, out_clients)


# speedup vs baseline: 1.4074x; 1.0061x over previous
import functools

import jax
import jax.numpy as jnp
from jax import lax
from jax.experimental import pallas as pl
from jax.experimental.pallas import tpu as pltpu
from jax.experimental.pallas import tpu_sc as plsc

N = 10000
NP = 10240
D = 128
E = 320000
CH = 128
DEG_N = 10240
SEG = 40

_f32 = jnp.float32
_MESH = plsc.VectorSubcoreMesh(core_axis_name="core", subcore_axis_name="subcore")


def _zero_rows(rows_v):
    def body(r, carry):
        for k in range(D // 16):
            rows_v[r, pl.ds(k * 16, 16)] = jnp.zeros((16,), _f32)
        return carry
    lax.fori_loop(0, CH, body, 0)


def _zero_acc(acc, rows_v, s):
    zb = pl.multiple_of(s * 640, 8)
    for i in range(5):
        pltpu.sync_copy(rows_v, acc.at[pl.ds(zb + i * 128, 128)])


def _writeback_acc(acc, rows_v, out_ref, s):
    @pl.when(s < 15)
    def _():
        zb = pl.multiple_of(s * 640, 8)
        for i in range(5):
            pltpu.sync_copy(acc.at[pl.ds(zb + i * 128, 128)], rows_v)
            pltpu.sync_copy(rows_v, out_ref.at[pl.ds(zb + i * 128, 128)])

    @pl.when(s == 15)
    def _():
        for i in range(3):
            pltpu.sync_copy(acc.at[pl.ds(9600 + i * 128, 128)], rows_v)
            pltpu.sync_copy(rows_v, out_ref.at[pl.ds(9600 + i * 128, 128)])
        pltpu.sync_copy(acc.at[pl.ds(9984, 16)], rows_v.at[pl.ds(0, 16)])
        pltpu.sync_copy(rows_v.at[pl.ds(0, 16)], out_ref.at[pl.ds(9984, 16)])


def _idx(slab, g):
    return slab.at[pl.ds(pl.multiple_of(g * CH, CH), CH)]


def _edge_pass(tbl_hbm, edge_hbm, acc, src_v, dst_v, r0, r1,
               g0s, g1s, base, segs):
    off = 0
    for L in segs:
        ne = L * CH
        b = pl.multiple_of(base + off, CH)
        off += ne
        pltpu.sync_copy(edge_hbm.at[0, pl.ds(b, ne)], src_v.at[pl.ds(0, ne)])
        pltpu.sync_copy(edge_hbm.at[1, pl.ds(b, ne)], dst_v.at[pl.ds(0, ne)])
        pltpu.async_copy(tbl_hbm.at[_idx(src_v, 0)], r0, g0s)

        def body(j, carry, L=L):
            i0 = 2 * j
            pltpu.async_copy(tbl_hbm.at[_idx(src_v, i0 + 1)], r1, g1s)
            pltpu.make_async_copy(tbl_hbm.at[_idx(src_v, i0)], r0, g0s).wait()
            pltpu.sync_copy(r0, acc.at[_idx(dst_v, i0)], add=True)

            @pl.when(i0 + 2 < L)
            def _():
                pltpu.async_copy(tbl_hbm.at[_idx(src_v, i0 + 2)], r0, g0s)
            pltpu.make_async_copy(tbl_hbm.at[_idx(src_v, i0 + 1)], r1, g1s).wait()
            pltpu.sync_copy(r1, acc.at[_idx(dst_v, i0 + 1)], add=True)
            return carry
        lax.fori_loop(0, L // 2, body, 0)


@functools.partial(
    pl.kernel,
    out_type=[jax.ShapeDtypeStruct((2, 1, DEG_N), _f32) for _ in range(6)],
    mesh=_MESH,
    scratch_types=[
        pltpu.VMEM((80 * CH,), jnp.int32),
        pltpu.VMEM((CH,), _f32),
        pltpu.VMEM((640,), _f32),
        pltpu.VMEM((640,), _f32),
        pltpu.SemaphoreType.DMA,
    ] + [pltpu.VMEM_SHARED((DEG_N,), _f32) for _ in range(6)],
)
def _deg_kernel(e0_hbm, e1_hbm, e2_hbm, o0, o1, o2, o3, o4, o5,
                slab, ones_v, zer_v, bnc_v, dsem, a0, a1, a2, a3, a4, a5):
    jobs = ((e0_hbm, 0, a0, o0), (e0_hbm, 1, a1, o1),
            (e1_hbm, 0, a2, o2), (e1_hbm, 1, a3, o3),
            (e2_hbm, 0, a4, o4), (e2_hbm, 1, a5, o5))
    c = lax.axis_index("core")
    s = lax.axis_index("subcore")
    wid = c * 16 + s
    for k in range(CH // 16):
        ones_v[pl.ds(k * 16, 16)] = jnp.ones((16,), _f32)
    for k in range(640 // 16):
        zer_v[pl.ds(k * 16, 16)] = jnp.zeros((16,), _f32)
    zd = pl.multiple_of(s * 640, 8)
    for _eh, _row, a, _o in jobs:
        pltpu.sync_copy(zer_v, a.at[pl.ds(zd, 640)])
    plsc.subcore_barrier()

    base = pl.multiple_of(wid * (80 * CH), CH)

    def scat(a, nch):
        def body(grp, carry):
            g0 = 4 * grp
            for k in range(4):
                pltpu.async_copy(ones_v, a.at[_idx(slab, g0 + k)], dsem,
                                 add=True)
            for k in range(4):
                pltpu.make_async_copy(ones_v, a.at[_idx(slab, g0 + k)],
                                      dsem).wait()
            return carry
        lax.fori_loop(0, nch // 4, body, 0)

    @pl.when(wid < 31)
    def _():
        for eh, row, a, _o in jobs:
            pltpu.sync_copy(eh.at[row, pl.ds(base, 80 * CH)], slab)
            scat(a, 80)

    @pl.when(wid == 31)
    def _():
        for eh, row, a, _o in jobs:
            pltpu.sync_copy(eh.at[row, pl.ds(317440, 20 * CH)],
                            slab.at[pl.ds(0, 20 * CH)])
            scat(a, 20)

    plsc.subcore_barrier()
    for _eh, _row, a, o in jobs:
        pltpu.sync_copy(a.at[pl.ds(zd, 640)], bnc_v)
        pltpu.sync_copy(bnc_v, o.at[c, 0, pl.ds(zd, 640)])


_SC_PASS_SCRATCH = [
    pltpu.VMEM((SEG * CH,), jnp.int32),
    pltpu.VMEM((SEG * CH,), jnp.int32),
    pltpu.VMEM((CH, D), _f32),
    pltpu.VMEM((CH, D), _f32),
    pltpu.VMEM_SHARED((NP, D), _f32),
    pltpu.SemaphoreType.DMA,
    pltpu.SemaphoreType.DMA,
]


@functools.partial(
    pl.kernel,
    out_type=[jax.ShapeDtypeStruct((N, D), _f32),
              jax.ShapeDtypeStruct((N, D), _f32)],
    mesh=_MESH,
    scratch_types=list(_SC_PASS_SCRATCH),
)
def _pass12_kernel(xs1_hbm, xs2_hbm, e1_hbm, e2_hbm,
                   p1_hbm, p2_hbm, src_v, dst_v, r0, r1, acc, g0s, g1s):
    c = lax.axis_index("core")
    s = lax.axis_index("subcore")
    _zero_rows(r0)
    _zero_acc(acc, r0, s)
    plsc.subcore_barrier()

    def run(tbl, eh):
        @pl.when(s < 15)
        def _():
            _edge_pass(tbl, eh, acc, src_v, dst_v, r0, r1, g0s, g1s,
                       pl.multiple_of(s * (160 * CH), CH),
                       (SEG, SEG, SEG, SEG))

        @pl.when(s == 15)
        def _():
            _edge_pass(tbl, eh, acc, src_v, dst_v, r0, r1, g0s, g1s,
                       2400 * CH, (SEG, SEG, 20))

    @pl.when(c == 0)
    def _():
        run(xs1_hbm, e1_hbm)

    @pl.when(c == 1)
    def _():
        run(xs2_hbm, e2_hbm)

    plsc.subcore_barrier()

    @pl.when(c == 0)
    def _():
        _writeback_acc(acc, r0, p1_hbm, s)

    @pl.when(c == 1)
    def _():
        _writeback_acc(acc, r0, p2_hbm, s)


@functools.partial(
    pl.kernel,
    out_type=jax.ShapeDtypeStruct((2, N, D), _f32),
    mesh=_MESH,
    scratch_types=list(_SC_PASS_SCRATCH),
)
def _pass0_kernel(xs0a_hbm, e0_hbm, out_hbm, src_v, dst_v,
                  r0, r1, acc, g0s, g1s):
    c = lax.axis_index("core")
    s = lax.axis_index("subcore")
    _zero_rows(r0)
    _zero_acc(acc, r0, s)
    plsc.subcore_barrier()
    wid = c * 16 + s

    def run(tbl):
        @pl.when(wid < 31)
        def _():
            _edge_pass(tbl, e0_hbm, acc, src_v, dst_v, r0, r1, g0s, g1s,
                       pl.multiple_of(wid * (80 * CH), CH), (SEG, SEG))

        @pl.when(wid == 31)
        def _():
            _edge_pass(tbl, e0_hbm, acc, src_v, dst_v, r0, r1, g0s, g1s,
                       317440, (20,))

    run(xs0a_hbm)
    plsc.subcore_barrier()
    _writeback_acc(acc, r0, out_hbm.at[c], s)


def _scales_body(d0, d1, d2, d3, d4, d5, out_ref):
    for j, d in enumerate((d0, d1, d2, d3, d4, d5)):
        out_ref[j] = lax.rsqrt(jnp.maximum(d[0, 0] + d[1, 0], 1.0))


def _xs_body(x_ref, s1_ref, s2_ref, o1_ref, o2_ref):
    x = x_ref[...]
    o1_ref[...] = x * s1_ref[...]
    o2_ref[...] = x * s2_ref[...]


def _mid_body(p1_ref, w11_ref, b11_ref, w21_ref, b21_ref, sd1_ref, ss0_ref,
              oc_ref, xs0a_ref):
    U = p1_ref[...]
    d1 = sd1_ref[...]
    h = jnp.maximum(jnp.dot(U, w11_ref[...], preferred_element_type=_f32) * d1
                    + b11_ref[...], 0.0)
    oc_ref[...] = (jnp.dot(U, w21_ref[...], preferred_element_type=_f32) * d1
                   + b21_ref[...])
    xs0a_ref[...] = h * ss0_ref[...]


def _out_body(p0_ref, p2_ref, w20_ref, b20_ref, w22_ref, b22_ref,
              sd0_ref, sd2_ref, o_ref):
    p0 = p0_ref[0] + p0_ref[1]
    o_ref[...] = (jnp.dot(p0, w20_ref[...], preferred_element_type=_f32)
                  * sd0_ref[...] + b20_ref[...]
                  + jnp.dot(p2_ref[...], w22_ref[...], preferred_element_type=_f32)
                  * sd2_ref[...] + b22_ref[...])


def kernel(x_loans, x_clients, edge_rel0, edge_rel1, edge_rel2,
           W1_0, b1_0, W1_1, b1_1, W1_2, b1_2,
           W2_0, b2_0, W2_1, b2_1, W2_2, b2_2):
    degp = _deg_kernel(edge_rel0, edge_rel1, edge_rel2)
    scal = pl.pallas_call(
        _scales_body,
        out_shape=jax.ShapeDtypeStruct((6, DEG_N), _f32))(*degp)
    ss0, sd0, ss1, sd1, ss2, sd2 = (scal[j, :N].reshape(N, 1) for j in range(6))

    xs1, xs2 = pl.pallas_call(
        _xs_body,
        out_shape=[jax.ShapeDtypeStruct((N, D), _f32)] * 2)(x_loans, ss1, ss2)

    P1, P2 = _pass12_kernel(xs1, xs2, edge_rel1, edge_rel2)

    out_clients, xs0a = pl.pallas_call(
        _mid_body,
        out_shape=[jax.ShapeDtypeStruct((N, D), _f32)] * 2)(
            P1, W1_1, b1_1.reshape(1, D), W2_1, b2_1.reshape(1, D), sd1, ss0)

    P0p = _pass0_kernel(xs0a, edge_rel0)

    out_loans = pl.pallas_call(
        _out_body,
        out_shape=jax.ShapeDtypeStruct((N, D), _f32))(
            P0p, P2, W2_0, b2_0.reshape(1, D), W2_2, b2_2.reshape(1, D), sd0, sd2)

    return (out_loans, out_clients)
